# baseline (speedup 1.0000x reference)
import functools

import jax
import jax.numpy as jnp
from jax import lax
from jax.experimental import pallas as pl
from jax.experimental.pallas import tpu as pltpu
from jax.experimental.pallas import tpu_sc as plsc


def kernel(user_indices, embedding_table):
    (B,) = user_indices.shape
    V, D = embedding_table.shape

    info = plsc.get_sparse_core_info()
    NC, NS = info.num_cores, info.num_subcores
    NW = NC * NS
    assert B % NW == 0
    b_per_w = B // NW

    mesh = plsc.VectorSubcoreMesh(core_axis_name="c", subcore_axis_name="s")

    @functools.partial(
        pl.kernel,
        mesh=mesh,
        out_type=jax.ShapeDtypeStruct((B, D), jnp.float32),
        scratch_types=[
            pltpu.VMEM((b_per_w,), jnp.int32),
            pltpu.VMEM((b_per_w, D), jnp.float32),
            pltpu.SemaphoreType.DMA,
        ],
        compiler_params=pltpu.CompilerParams(use_tc_tiling_on_sc=False),
    )
    def gather_kernel(idx_hbm, table_hbm, out_hbm, idx_v, rows_v, sem):
        wid = lax.axis_index("s") * NC + lax.axis_index("c")
        base = wid * b_per_w
        pltpu.sync_copy(idx_hbm.at[pl.ds(base, b_per_w)], idx_v)
        pltpu.async_copy(table_hbm.at[idx_v], rows_v, sem).wait()
        pltpu.sync_copy(rows_v, out_hbm.at[pl.ds(base, b_per_w)])

    return gather_kernel(user_indices.astype(jnp.int32), embedding_table)


# speedup vs baseline: 2.5617x; 2.5617x over previous
import functools

import jax
import jax.numpy as jnp
from jax import lax
from jax.experimental import pallas as pl
from jax.experimental.pallas import tpu as pltpu
from jax.experimental.pallas import tpu_sc as plsc

_I32 = jnp.int32


def kernel(user_indices, embedding_table):
    (B,) = user_indices.shape
    V, D = embedding_table.shape

    tab_t = embedding_table.T

    info = plsc.get_sparse_core_info()
    NC, NS = info.num_cores, info.num_subcores
    NW = NC * NS
    L = 16

    CH = 512
    CPW = (V // CH) // NW
    MAIN = CPW * CH
    n_full_extra = (V - MAIN * NW) // CH
    tail = V - MAIN * NW - n_full_extra * CH
    assert MAIN % 128 == 0 and CH % 128 == 0
    assert n_full_extra == 1 and 0 < tail <= 64 and B % 1024 == 0

    extra_lo = MAIN * NW
    tail_lo = extra_lo + CH
    NIB = B // 1024
    SR = 128
    FLUSH_AT = SR - L

    mesh = plsc.VectorSubcoreMesh(core_axis_name="c", subcore_axis_name="s")

    @functools.partial(
        pl.kernel,
        mesh=mesh,
        out_type=jax.ShapeDtypeStruct((B, 128), jnp.float32),
        scratch_types=[
            pltpu.VMEM((1024,), _I32),
            pltpu.VMEM((1024,), _I32),
            pltpu.VMEM((B,), _I32),
            pltpu.VMEM((B,), _I32),
            pltpu.VMEM((D, CH), jnp.float32),
            pltpu.VMEM((D, CH), jnp.float32),
            pltpu.VMEM((D, 64), jnp.float32),
            pltpu.VMEM((SR, 128), jnp.float32),
            pltpu.VMEM((SR,), _I32),
            pltpu.SemaphoreType.DMA,
            pltpu.SemaphoreType.DMA,
            pltpu.SemaphoreType.DMA,
            pltpu.SemaphoreType.DMA,
        ],
        compiler_params=pltpu.CompilerParams(
            use_tc_tiling_on_sc=True, needs_layout_passes=False
        ),
    )
    def scan_kernel(
        idx_hbm, tab_hbm, out_hbm,
        ibuf0, ibuf1, hits_u, hits_pos, buf0, buf1, tailbuf,
        stage, stage_pos, sem_i, sem0, sem1, sem_out,
    ):
        wid = lax.axis_index("s") * NC + lax.axis_index("c")
        iota = lax.broadcasted_iota(_I32, (L,), 0)
        neg1 = jnp.full((L,), -1, _I32)

        def _pf(j, c):
            hits_u[pl.ds(j * L, L)] = neg1
            return c

        lax.fori_loop(0, B // L, _pf, 0)
        for r in range(SR // L):
            stage_pos[pl.ds(r * L, L)] = neg1

        lo = wid * MAIN
        far = jnp.asarray(2**30, _I32)
        elo = jnp.where(wid == 0, extra_lo, far)
        tlo = jnp.where(wid == 1, tail_lo, far)

        pltpu.async_copy(tab_hbm.at[:, pl.ds(lo, CH)], buf0, sem0)
        pltpu.async_copy(tab_hbm.at[:, pl.ds(lo + CH, CH)], buf1, sem1)

        nh = _I32(0)
        descs = [None, None]
        descs[0] = pltpu.async_copy(idx_hbm.at[pl.ds(0, 1024)], ibuf0, sem_i)
        for ib in range(NIB):
            cur = ibuf0 if ib % 2 == 0 else ibuf1
            nxt = ibuf1 if ib % 2 == 0 else ibuf0
            descs[ib % 2].wait()
            if ib + 1 < NIB:
                descs[(ib + 1) % 2] = pltpu.async_copy(
                    idx_hbm.at[pl.ds((ib + 1) * 1024, 1024)], nxt, sem_i
                )

            def _scan(j, cursor, cur=cur, ib=ib):
                v = cur[pl.ds(j * L, L)]
                pos = iota + (ib * 1024) + j * L
                m = (
                    ((v >= lo) & (v < lo + MAIN))
                    | ((v >= elo) & (v < elo + CH))
                    | (v >= tlo)
                )
                offs = plsc.cumsum(jnp.where(m, 1, 0).astype(_I32))
                slots = cursor + offs - 1
                plsc.store_scatter(hits_u, [slots], v, mask=m)
                plsc.store_scatter(hits_pos, [slots], pos, mask=m)
                return cursor + offs[15]

            nh = lax.fori_loop(0, 1024 // L, _scan, nh)

        nblocks = (nh + (L - 1)) // L

        def _flush():
            pltpu.async_copy(
                stage,
                out_hbm.at[plsc.Indices(stage_pos, ignored_value=-1)],
                sem_out,
            ).wait()
            for r in range(SR // L):
                stage_pos[pl.ds(r * L, L)] = neg1

        def _extract(buf, c0u, width, cursor):
            def _blk(j, cur):
                vu = hits_u[pl.ds(j * L, L)]
                m = (vu >= c0u) & (vu < c0u + width)
                offs = plsc.cumsum(jnp.where(m, 1, 0).astype(_I32))
                tot = offs[15]

                @pl.when(tot > 0)
                def _():
                    vp = hits_pos[pl.ds(j * L, L)]
                    vul = jnp.where(m, vu - c0u, 0)
                    slots = cur + offs - 1
                    for f in range(D):
                        fv = jnp.full((L,), f, _I32)
                        val = plsc.load_gather(buf, [fv, vul])
                        plsc.store_scatter(stage, [slots, fv], val, mask=m)
                    plsc.store_scatter(stage_pos, [slots], vp, mask=m)

                cur2 = cur + tot

                @pl.when(cur2 >= FLUSH_AT)
                def _():
                    _flush()

                return jnp.where(cur2 >= FLUSH_AT, 0, cur2)

            return lax.fori_loop(0, nblocks, _blk, cursor)

        def _pair(k, cursor):
            ca = lo + (2 * k) * CH
            pltpu.make_async_copy(tab_hbm.at[:, pl.ds(ca, CH)], buf0, sem0).wait()
            cursor = _extract(buf0, ca, CH, cursor)

            @pl.when(2 * k + 2 < CPW)
            def _():
                pltpu.async_copy(
                    tab_hbm.at[:, pl.ds(ca + 2 * CH, CH)], buf0, sem0
                )

            pltpu.make_async_copy(
                tab_hbm.at[:, pl.ds(ca + CH, CH)], buf1, sem1
            ).wait()
            cursor = _extract(buf1, ca + CH, CH, cursor)

            @pl.when(2 * k + 3 < CPW)
            def _():
                pltpu.async_copy(
                    tab_hbm.at[:, pl.ds(ca + 3 * CH, CH)], buf1, sem1
                )

            return cursor

        cursor = lax.fori_loop(0, CPW // 2, _pair, nh * 0)
        if CPW % 2 == 1:
            clast = lo + (CPW - 1) * CH
            pltpu.make_async_copy(
                tab_hbm.at[:, pl.ds(clast, CH)], buf0, sem0
            ).wait()
            cursor = _extract(buf0, clast, CH, cursor)

        pltpu.sync_copy(tab_hbm.at[:, pl.ds(extra_lo, CH)], buf1)
        cursor = _extract(buf1, elo, CH, cursor)
        pltpu.sync_copy(tab_hbm.at[:, pl.ds(tail_lo, tail)], tailbuf.at[:, :tail])
        cursor = _extract(tailbuf, tlo, tail, cursor)

        _flush()

    out128 = scan_kernel(user_indices.astype(_I32), tab_t)
    return out128[:, :D]


# speedup vs baseline: 3.7265x; 1.4547x over previous
import functools

import jax
import jax.numpy as jnp
from jax import lax
from jax.experimental import pallas as pl
from jax.experimental.pallas import tpu as pltpu
from jax.experimental.pallas import tpu_sc as plsc

_I32 = jnp.int32


def kernel(user_indices, embedding_table):
    (B,) = user_indices.shape
    V, D = embedding_table.shape

    tab_t = embedding_table.T

    info = plsc.get_sparse_core_info()
    NC, NS = info.num_cores, info.num_subcores
    NW = NC * NS
    L = 16

    CH = 512
    CPW = (V // CH) // NW
    MAIN = CPW * CH
    n_full_extra = (V - MAIN * NW) // CH
    tail = V - MAIN * NW - n_full_extra * CH
    assert MAIN % 128 == 0 and CH % 128 == 0
    assert n_full_extra == 1 and 0 < tail <= 64 and B % 1024 == 0

    extra_lo = MAIN * NW
    tail_lo = extra_lo + CH
    NIB = B // 1024
    SR = 128
    FLUSH_AT = SR - L

    mesh = plsc.VectorSubcoreMesh(core_axis_name="c", subcore_axis_name="s")

    @functools.partial(
        pl.kernel,
        mesh=mesh,
        out_type=jax.ShapeDtypeStruct((B, 128), jnp.float32),
        scratch_types=[
            pltpu.VMEM((1024,), _I32),
            pltpu.VMEM((1024,), _I32),
            pltpu.VMEM((B,), _I32),
            pltpu.VMEM((B,), _I32),
            pltpu.VMEM((D, CH), jnp.float32),
            pltpu.VMEM((D, CH), jnp.float32),
            pltpu.VMEM((D, 64), jnp.float32),
            pltpu.VMEM((SR, 128), jnp.float32),
            pltpu.VMEM((SR,), _I32),
            pltpu.VMEM((1024,), _I32),
            pltpu.SemaphoreType.DMA,
            pltpu.SemaphoreType.DMA,
            pltpu.SemaphoreType.DMA,
            pltpu.SemaphoreType.DMA,
        ],
        compiler_params=pltpu.CompilerParams(
            use_tc_tiling_on_sc=True, needs_layout_passes=False
        ),
    )
    def scan_kernel(
        idx_hbm, tab_hbm, out_hbm,
        ibuf0, ibuf1, hits_u, hits_pos, buf0, buf1, tailbuf,
        stage, stage_pos, loc, sem_i, sem0, sem1, sem_out,
    ):
        wid = lax.axis_index("s") * NC + lax.axis_index("c")
        iota = lax.broadcasted_iota(_I32, (L,), 0)
        neg1 = jnp.full((L,), -1, _I32)

        def _pf(j, c):
            hits_u[pl.ds(j * L, L)] = neg1
            return c

        lax.fori_loop(0, B // L, _pf, 0)
        for r in range(SR // L):
            stage_pos[pl.ds(r * L, L)] = neg1

        lo = wid * MAIN
        far = jnp.asarray(2**30, _I32)
        elo = jnp.where(wid == 0, extra_lo, far)
        tlo = jnp.where(wid == 1, tail_lo, far)

        pltpu.async_copy(tab_hbm.at[:, pl.ds(lo, CH)], buf0, sem0)
        pltpu.async_copy(tab_hbm.at[:, pl.ds(lo + CH, CH)], buf1, sem1)

        nh = _I32(0)
        descs = [None, None]
        descs[0] = pltpu.async_copy(idx_hbm.at[pl.ds(0, 1024)], ibuf0, sem_i)
        for ib in range(NIB):
            cur = ibuf0 if ib % 2 == 0 else ibuf1
            nxt = ibuf1 if ib % 2 == 0 else ibuf0
            descs[ib % 2].wait()
            if ib + 1 < NIB:
                descs[(ib + 1) % 2] = pltpu.async_copy(
                    idx_hbm.at[pl.ds((ib + 1) * 1024, 1024)], nxt, sem_i
                )

            def _scan(j, cursor, cur=cur, ib=ib):
                v = cur[pl.ds(j * L, L)]
                pos = iota + (ib * 1024) + j * L
                m = (
                    ((v >= lo) & (v < lo + MAIN))
                    | ((v >= elo) & (v < elo + CH))
                    | (v >= tlo)
                )
                offs = plsc.cumsum(jnp.where(m, 1, 0).astype(_I32))
                slots = cursor + offs - 1
                plsc.store_scatter(hits_u, [slots], v, mask=m)
                plsc.store_scatter(hits_pos, [slots], pos, mask=m)
                return cursor + offs[15]

            nh = lax.fori_loop(0, 1024 // L, _scan, nh)

        nblocks = (nh + (L - 1)) // L

        def _flush():
            pltpu.async_copy(
                stage,
                out_hbm.at[plsc.Indices(stage_pos, ignored_value=-1)],
                sem_out,
            ).wait()
            for r in range(SR // L):
                stage_pos[pl.ds(r * L, L)] = neg1

        LOC = 1024

        def _extract(buf, c0u, width, cursor):
            def _passA(P):
                def _blkA(j, cnt):
                    vu = hits_u[pl.ds(j * L, L)]
                    m = (vu >= c0u) & (vu < c0u + width)
                    offs = plsc.cumsum(jnp.where(m, 1, 0).astype(_I32))
                    tot = offs[15]

                    @pl.when(tot > 0)
                    def _():
                        vp = hits_pos[pl.ds(j * L, L)]
                        packed = jnp.where(m, vu - c0u, 0) | (vp << 10)
                        gr = cnt + offs - 1
                        sel = m & (gr >= P) & (gr < P + LOC)
                        plsc.store_scatter(loc, [gr - P], packed, mask=sel)

                    return cnt + tot

                return lax.fori_loop(0, nblocks, _blkA, jnp.int32(0))

            def _process(n_this, cursor):
                def _blkB(j, cur):
                    packed = loc[pl.ds(j * L, L)]
                    m2 = (iota + j * L) < n_this
                    vul = jnp.where(m2, packed & (LOC - 1), 0)
                    vp = packed >> 10
                    slots = cur + iota
                    for f in range(D):
                        fv = jnp.full((L,), f, _I32)
                        val = plsc.load_gather(buf, [fv, vul])
                        plsc.store_scatter(stage, [slots, fv], val, mask=m2)
                    plsc.store_scatter(stage_pos, [slots], vp, mask=m2)
                    cur2 = cur + jnp.minimum(n_this - j * L, L)

                    @pl.when(cur2 >= FLUSH_AT)
                    def _():
                        _flush()

                    return jnp.where(cur2 >= FLUSH_AT, 0, cur2)

                return lax.fori_loop(0, (n_this + (L - 1)) // L, _blkB, cursor)

            n_chunk = _passA(jnp.int32(0))
            cursor = _process(jnp.minimum(n_chunk, LOC), cursor)

            def _more(st):
                P, cur = st
                _passA(P)
                cur = _process(jnp.minimum(n_chunk - P, LOC), cur)
                return (P + LOC, cur)

            _, cursor = lax.while_loop(
                lambda st: st[0] < n_chunk, _more, (jnp.int32(LOC), cursor)
            )
            return cursor

        def _pair(k, cursor):
            ca = lo + (2 * k) * CH
            pltpu.make_async_copy(tab_hbm.at[:, pl.ds(ca, CH)], buf0, sem0).wait()
            cursor = _extract(buf0, ca, CH, cursor)

            @pl.when(2 * k + 2 < CPW)
            def _():
                pltpu.async_copy(
                    tab_hbm.at[:, pl.ds(ca + 2 * CH, CH)], buf0, sem0
                )

            pltpu.make_async_copy(
                tab_hbm.at[:, pl.ds(ca + CH, CH)], buf1, sem1
            ).wait()
            cursor = _extract(buf1, ca + CH, CH, cursor)

            @pl.when(2 * k + 3 < CPW)
            def _():
                pltpu.async_copy(
                    tab_hbm.at[:, pl.ds(ca + 3 * CH, CH)], buf1, sem1
                )

            return cursor

        cursor = lax.fori_loop(0, CPW // 2, _pair, nh * 0)
        if CPW % 2 == 1:
            clast = lo + (CPW - 1) * CH
            pltpu.make_async_copy(
                tab_hbm.at[:, pl.ds(clast, CH)], buf0, sem0
            ).wait()
            cursor = _extract(buf0, clast, CH, cursor)

        pltpu.sync_copy(tab_hbm.at[:, pl.ds(extra_lo, CH)], buf1)
        cursor = _extract(buf1, elo, CH, cursor)
        pltpu.sync_copy(tab_hbm.at[:, pl.ds(tail_lo, tail)], tailbuf.at[:, :tail])
        cursor = _extract(tailbuf, tlo, tail, cursor)

        _flush()

    out128 = scan_kernel(user_indices.astype(_I32), tab_t)
    return out128[:, :D]


# speedup vs baseline: 3.7691x; 1.0114x over previous
import functools

import jax
import jax.numpy as jnp
from jax import lax
from jax.experimental import pallas as pl
from jax.experimental.pallas import tpu as pltpu
from jax.experimental.pallas import tpu_sc as plsc

_I32 = jnp.int32


def kernel(user_indices, embedding_table):
    (B,) = user_indices.shape
    V, D = embedding_table.shape

    tab_t = embedding_table.T

    info = plsc.get_sparse_core_info()
    NC, NS = info.num_cores, info.num_subcores
    NW = NC * NS
    L = 16

    CH = 512
    CPW = (V // CH) // NW
    MAIN = CPW * CH
    n_full_extra = (V - MAIN * NW) // CH
    tail = V - MAIN * NW - n_full_extra * CH
    assert MAIN % 128 == 0 and CH % 128 == 0
    assert n_full_extra == 1 and 0 < tail <= 64 and B % 1024 == 0

    extra_lo = MAIN * NW
    tail_lo = extra_lo + CH
    NIB = B // 1024
    SR = 128
    FLUSH_AT = SR - L

    mesh = plsc.VectorSubcoreMesh(core_axis_name="c", subcore_axis_name="s")

    @functools.partial(
        pl.kernel,
        mesh=mesh,
        out_type=jax.ShapeDtypeStruct((B, 128), jnp.float32),
        scratch_types=[
            pltpu.VMEM((1024,), _I32),
            pltpu.VMEM((1024,), _I32),
            pltpu.VMEM((B,), _I32),
            pltpu.VMEM((B,), _I32),
            pltpu.VMEM((D, CH), jnp.float32),
            pltpu.VMEM((D, CH), jnp.float32),
            pltpu.VMEM((D, 64), jnp.float32),
            pltpu.VMEM((SR, 128), jnp.float32),
            pltpu.VMEM((SR,), _I32),
            pltpu.VMEM((1024,), _I32),
            pltpu.SemaphoreType.DMA,
            pltpu.SemaphoreType.DMA,
            pltpu.SemaphoreType.DMA,
            pltpu.SemaphoreType.DMA,
        ],
        compiler_params=pltpu.CompilerParams(
            use_tc_tiling_on_sc=True, needs_layout_passes=False
        ),
    )
    def scan_kernel(
        idx_hbm, tab_hbm, out_hbm,
        ibuf0, ibuf1, hits_u, hits_pos, buf0, buf1, tailbuf,
        stage, stage_pos, loc, sem_i, sem0, sem1, sem_out,
    ):
        wid = lax.axis_index("s") * NC + lax.axis_index("c")
        iota = lax.broadcasted_iota(_I32, (L,), 0)
        neg1 = jnp.full((L,), -1, _I32)

        def _pf(j, c):
            hits_u[pl.ds(j * L, L)] = neg1
            return c

        lax.fori_loop(0, B // L, _pf, 0)
        for r in range(SR // L):
            stage_pos[pl.ds(r * L, L)] = neg1

        lo = wid * MAIN
        far = jnp.asarray(2**30, _I32)
        elo = jnp.where(wid == 0, extra_lo, far)
        tlo = jnp.where(wid == 1, tail_lo, far)

        pltpu.async_copy(tab_hbm.at[:, pl.ds(lo, CH)], buf0, sem0)
        pltpu.async_copy(tab_hbm.at[:, pl.ds(lo + CH, CH)], buf1, sem1)

        nh = _I32(0)
        descs = [None, None]
        descs[0] = pltpu.async_copy(idx_hbm.at[pl.ds(0, 1024)], ibuf0, sem_i)
        for ib in range(NIB):
            cur = ibuf0 if ib % 2 == 0 else ibuf1
            nxt = ibuf1 if ib % 2 == 0 else ibuf0
            descs[ib % 2].wait()
            if ib + 1 < NIB:
                descs[(ib + 1) % 2] = pltpu.async_copy(
                    idx_hbm.at[pl.ds((ib + 1) * 1024, 1024)], nxt, sem_i
                )

            def _scan(j, cursor, cur=cur, ib=ib):
                v = cur[pl.ds(j * L, L)]
                pos = iota + (ib * 1024) + j * L
                m = (
                    ((v >= lo) & (v < lo + MAIN))
                    | ((v >= elo) & (v < elo + CH))
                    | (v >= tlo)
                )
                offs = plsc.cumsum(jnp.where(m, 1, 0).astype(_I32))
                slots = cursor + offs - 1
                plsc.store_scatter(hits_u, [slots], v, mask=m)
                plsc.store_scatter(hits_pos, [slots], pos, mask=m)
                return cursor + offs[15]

            nh = lax.fori_loop(0, 1024 // L, _scan, nh)

        nblocks = (nh + (L - 1)) // L

        def _flush():
            pltpu.async_copy(
                stage,
                out_hbm.at[plsc.Indices(stage_pos, ignored_value=-1)],
                sem_out,
            ).wait()
            for r in range(SR // L):
                stage_pos[pl.ds(r * L, L)] = neg1

        LOC = 1024

        def _extract(buf, c0u, width, cursor):
            def _passA(P):
                def _blkA(j, cnt):
                    vu = hits_u[pl.ds(j * L, L)]
                    m = (vu >= c0u) & (vu < c0u + width)
                    offs = plsc.cumsum(jnp.where(m, 1, 0).astype(_I32))
                    tot = offs[15]

                    @pl.when(tot > 0)
                    def _():
                        vp = hits_pos[pl.ds(j * L, L)]
                        packed = jnp.where(m, vu - c0u, 0) | (vp << 10)
                        gr = cnt + offs - 1
                        sel = m & (gr >= P) & (gr < P + LOC)
                        plsc.store_scatter(loc, [gr - P], packed, mask=sel)

                    return cnt + tot

                return lax.fori_loop(0, nblocks, _blkA, jnp.int32(0))

            def _process(n_this, cursor):
                def _blkB(j, cur):
                    packed = loc[pl.ds(j * L, L)]
                    m2 = (iota + j * L) < n_this
                    vul = jnp.where(m2, packed & (LOC - 1), 0)
                    vp = packed >> 10
                    slots = cur + iota
                    for f in range(D):
                        fv = jnp.full((L,), f, _I32)
                        val = plsc.load_gather(buf, [fv, vul])
                        plsc.store_scatter(stage, [slots, fv], val, mask=m2)
                    plsc.store_scatter(stage_pos, [slots], vp, mask=m2)
                    cur2 = cur + jnp.minimum(n_this - j * L, L)

                    @pl.when(cur2 >= FLUSH_AT)
                    def _():
                        _flush()

                    return jnp.where(cur2 >= FLUSH_AT, 0, cur2)

                return lax.fori_loop(0, (n_this + (L - 1)) // L, _blkB, cursor)

            n_chunk = _passA(jnp.int32(0))
            cursor = _process(jnp.minimum(n_chunk, LOC), cursor)

            def _more(st):
                P, cur = st
                _passA(P)
                cur = _process(jnp.minimum(n_chunk - P, LOC), cur)
                return (P + LOC, cur)

            _, cursor = lax.while_loop(
                lambda st: st[0] < n_chunk, _more, (jnp.int32(LOC), cursor)
            )
            return cursor

        def _pair(k, cursor):
            ca = lo + (2 * k) * CH
            pltpu.make_async_copy(tab_hbm.at[:, pl.ds(ca, CH)], buf0, sem0).wait()
            cursor = _extract(buf0, ca, CH, cursor)

            @pl.when(2 * k + 2 < CPW)
            def _():
                pltpu.async_copy(
                    tab_hbm.at[:, pl.ds(ca + 2 * CH, CH)], buf0, sem0
                )

            pltpu.make_async_copy(
                tab_hbm.at[:, pl.ds(ca + CH, CH)], buf1, sem1
            ).wait()
            cursor = _extract(buf1, ca + CH, CH, cursor)

            @pl.when(2 * k + 3 < CPW)
            def _():
                pltpu.async_copy(
                    tab_hbm.at[:, pl.ds(ca + 3 * CH, CH)], buf1, sem1
                )

            return cursor

        cursor = lax.fori_loop(0, CPW // 2, _pair, jnp.int32(0))
        if CPW % 2 == 1:
            clast = lo + (CPW - 1) * CH
            pltpu.make_async_copy(
                tab_hbm.at[:, pl.ds(clast, CH)], buf0, sem0
            ).wait()
            cursor = _extract(buf0, clast, CH, cursor)

        @pl.when(wid == 0)
        def _():
            pltpu.sync_copy(tab_hbm.at[:, pl.ds(extra_lo, CH)], buf1)

        cursor = _extract(buf1, elo, CH, cursor)

        @pl.when(wid == 1)
        def _():
            pltpu.sync_copy(
                tab_hbm.at[:, pl.ds(tail_lo, tail)], tailbuf.at[:, :tail]
            )

        cursor = _extract(tailbuf, tlo, tail, cursor)

        _flush()

    out128 = scan_kernel(user_indices.astype(_I32), tab_t)
    return out128[:, :D]


# speedup vs baseline: 3.8808x; 1.0296x over previous
import functools

import jax
import jax.numpy as jnp
from jax import lax
from jax.experimental import pallas as pl
from jax.experimental.pallas import tpu as pltpu
from jax.experimental.pallas import tpu_sc as plsc

_I32 = jnp.int32


def kernel(user_indices, embedding_table):
    (B,) = user_indices.shape
    V, D = embedding_table.shape

    tab_t = embedding_table.T

    info = plsc.get_sparse_core_info()
    NC, NS = info.num_cores, info.num_subcores
    NW = NC * NS
    L = 16

    CH = 512
    CPW = (V // CH) // NW
    MAIN = CPW * CH
    n_full_extra = (V - MAIN * NW) // CH
    tail = V - MAIN * NW - n_full_extra * CH
    assert MAIN % 128 == 0 and CH % 128 == 0
    assert n_full_extra == 1 and 0 < tail <= 64 and B % 1024 == 0

    extra_lo = MAIN * NW
    tail_lo = extra_lo + CH
    NIB = B // 1024
    SR = 160
    FLUSH_AT = SR - L

    mesh = plsc.VectorSubcoreMesh(core_axis_name="c", subcore_axis_name="s")

    @functools.partial(
        pl.kernel,
        mesh=mesh,
        out_type=jax.ShapeDtypeStruct((B, 128), jnp.float32),
        scratch_types=[
            pltpu.VMEM((1024,), _I32),
            pltpu.VMEM((1024,), _I32),
            pltpu.VMEM((B,), _I32),
            pltpu.VMEM((B,), _I32),
            pltpu.VMEM((D, CH), jnp.float32),
            pltpu.VMEM((D, CH), jnp.float32),
            pltpu.VMEM((D, 64), jnp.float32),
            pltpu.VMEM((SR, 128), jnp.float32),
            pltpu.VMEM((SR,), _I32),
            pltpu.VMEM((1024,), _I32),
            pltpu.SemaphoreType.DMA,
            pltpu.SemaphoreType.DMA,
            pltpu.SemaphoreType.DMA,
            pltpu.SemaphoreType.DMA,
        ],
        compiler_params=pltpu.CompilerParams(
            use_tc_tiling_on_sc=True, needs_layout_passes=False
        ),
    )
    def scan_kernel(
        idx_hbm, tab_hbm, out_hbm,
        ibuf0, ibuf1, hits_u, hits_pos, buf0, buf1, tailbuf,
        stage, stage_pos, loc, sem_i, sem0, sem1, sem_out,
    ):
        wid = lax.axis_index("s") * NC + lax.axis_index("c")
        iota = lax.broadcasted_iota(_I32, (L,), 0)
        neg1 = jnp.full((L,), -1, _I32)

        for r in range(SR // L):
            stage_pos[pl.ds(r * L, L)] = neg1

        lo = wid * MAIN
        far = jnp.asarray(2**30, _I32)
        elo = jnp.where(wid == 0, extra_lo, far)
        tlo = jnp.where(wid == 1, tail_lo, far)

        pltpu.async_copy(tab_hbm.at[:, pl.ds(lo, CH)], buf0, sem0)
        pltpu.async_copy(tab_hbm.at[:, pl.ds(lo + CH, CH)], buf1, sem1)

        nh = _I32(0)
        descs = [None, None]
        descs[0] = pltpu.async_copy(idx_hbm.at[pl.ds(0, 1024)], ibuf0, sem_i)
        for ib in range(NIB):
            cur = ibuf0 if ib % 2 == 0 else ibuf1
            nxt = ibuf1 if ib % 2 == 0 else ibuf0
            descs[ib % 2].wait()
            if ib + 1 < NIB:
                descs[(ib + 1) % 2] = pltpu.async_copy(
                    idx_hbm.at[pl.ds((ib + 1) * 1024, 1024)], nxt, sem_i
                )

            def _scan(j, cursor, cur=cur, ib=ib):
                v = cur[pl.ds(j * L, L)]
                pos = iota + (ib * 1024) + j * L
                m = (
                    ((v >= lo) & (v < lo + MAIN))
                    | ((v >= elo) & (v < elo + CH))
                    | (v >= tlo)
                )
                offs = plsc.cumsum(jnp.where(m, 1, 0).astype(_I32))
                slots = cursor + offs - 1
                plsc.store_scatter(hits_u, [slots], v, mask=m)
                plsc.store_scatter(hits_pos, [slots], pos, mask=m)
                return cursor + offs[15]

            nh = lax.fori_loop(0, 1024 // L, _scan, nh)

        plsc.store_scatter(hits_u, [nh + iota], neg1, mask=(nh + iota) < B)

        nblocks = (nh + (L - 1)) // L

        def _flush():
            pltpu.async_copy(
                stage,
                out_hbm.at[plsc.Indices(stage_pos, ignored_value=-1)],
                sem_out,
            ).wait()
            for r in range(SR // L):
                stage_pos[pl.ds(r * L, L)] = neg1

        LOC = 1024

        def _extract(buf, c0u, width, cursor):
            def _passA(P):
                def _blkA(j, cnt):
                    vu = hits_u[pl.ds(j * L, L)]
                    m = (vu >= c0u) & (vu < c0u + width)
                    offs = plsc.cumsum(jnp.where(m, 1, 0).astype(_I32))
                    tot = offs[15]

                    @pl.when(tot > 0)
                    def _():
                        vp = hits_pos[pl.ds(j * L, L)]
                        packed = jnp.where(m, vu - c0u, 0) | (vp << 10)
                        gr = cnt + offs - 1
                        sel = m & (gr >= P) & (gr < P + LOC)
                        plsc.store_scatter(loc, [gr - P], packed, mask=sel)

                    return cnt + tot

                return lax.fori_loop(0, nblocks, _blkA, jnp.int32(0))

            def _process(n_this, cursor):
                def _blkB(j, cur):
                    packed = loc[pl.ds(j * L, L)]
                    m2 = (iota + j * L) < n_this
                    vul = jnp.where(m2, packed & (LOC - 1), 0)
                    vp = packed >> 10
                    slots = cur + iota
                    for f in range(D):
                        fv = jnp.full((L,), f, _I32)
                        val = plsc.load_gather(buf, [fv, vul])
                        plsc.store_scatter(stage, [slots, fv], val, mask=m2)
                    plsc.store_scatter(stage_pos, [slots], vp, mask=m2)
                    cur2 = cur + jnp.minimum(n_this - j * L, L)

                    @pl.when(cur2 >= FLUSH_AT)
                    def _():
                        _flush()

                    return jnp.where(cur2 >= FLUSH_AT, 0, cur2)

                return lax.fori_loop(0, (n_this + (L - 1)) // L, _blkB, cursor)

            n_chunk = _passA(jnp.int32(0))
            cursor = _process(jnp.minimum(n_chunk, LOC), cursor)

            def _more(st):
                P, cur = st
                _passA(P)
                cur = _process(jnp.minimum(n_chunk - P, LOC), cur)
                return (P + LOC, cur)

            _, cursor = lax.while_loop(
                lambda st: st[0] < n_chunk, _more, (jnp.int32(LOC), cursor)
            )
            return cursor

        def _pair(k, cursor):
            ca = lo + (2 * k) * CH
            pltpu.make_async_copy(tab_hbm.at[:, pl.ds(ca, CH)], buf0, sem0).wait()
            cursor = _extract(buf0, ca, CH, cursor)

            @pl.when(2 * k + 2 < CPW)
            def _():
                pltpu.async_copy(
                    tab_hbm.at[:, pl.ds(ca + 2 * CH, CH)], buf0, sem0
                )

            pltpu.make_async_copy(
                tab_hbm.at[:, pl.ds(ca + CH, CH)], buf1, sem1
            ).wait()
            cursor = _extract(buf1, ca + CH, CH, cursor)

            @pl.when(2 * k + 3 < CPW)
            def _():
                pltpu.async_copy(
                    tab_hbm.at[:, pl.ds(ca + 3 * CH, CH)], buf1, sem1
                )

            return cursor

        cursor = lax.fori_loop(0, CPW // 2, _pair, jnp.int32(0))
        if CPW % 2 == 1:
            clast = lo + (CPW - 1) * CH
            pltpu.make_async_copy(
                tab_hbm.at[:, pl.ds(clast, CH)], buf0, sem0
            ).wait()
            cursor = _extract(buf0, clast, CH, cursor)

        @pl.when(wid == 0)
        def _():
            pltpu.sync_copy(tab_hbm.at[:, pl.ds(extra_lo, CH)], buf1)

        cursor = _extract(buf1, elo, CH, cursor)

        @pl.when(wid == 1)
        def _():
            pltpu.sync_copy(
                tab_hbm.at[:, pl.ds(tail_lo, tail)], tailbuf.at[:, :tail]
            )

        cursor = _extract(tailbuf, tlo, tail, cursor)

        _flush()

    out128 = scan_kernel(user_indices.astype(_I32), tab_t)
    return out128[:, :D]


# speedup vs baseline: 3.9112x; 1.0078x over previous
import functools

import jax
import jax.numpy as jnp
from jax import lax
from jax.experimental import pallas as pl
from jax.experimental.pallas import tpu as pltpu
from jax.experimental.pallas import tpu_sc as plsc

_I32 = jnp.int32


def kernel(user_indices, embedding_table):
    (B,) = user_indices.shape
    V, D = embedding_table.shape

    tab_t = embedding_table.T

    info = plsc.get_sparse_core_info()
    NC, NS = info.num_cores, info.num_subcores
    NW = NC * NS
    L = 16

    CH = 512
    CPW = (V // CH) // NW
    MAIN = CPW * CH
    n_full_extra = (V - MAIN * NW) // CH
    tail = V - MAIN * NW - n_full_extra * CH
    assert MAIN % 128 == 0 and CH % 128 == 0
    assert n_full_extra == 1 and 0 < tail <= 64 and B % 1024 == 0
    assert NW >= 5 and CH % 4 == 0

    extra_lo = MAIN * NW
    tail_lo = extra_lo + CH
    NIB = B // 1024
    SR = 160
    FLUSH_AT = SR - L

    mesh = plsc.VectorSubcoreMesh(core_axis_name="c", subcore_axis_name="s")

    @functools.partial(
        pl.kernel,
        mesh=mesh,
        out_type=jax.ShapeDtypeStruct((B, 128), jnp.float32),
        scratch_types=[
            pltpu.VMEM((1024,), _I32),
            pltpu.VMEM((1024,), _I32),
            pltpu.VMEM((B,), _I32),
            pltpu.VMEM((B,), _I32),
            pltpu.VMEM((D, CH), jnp.float32),
            pltpu.VMEM((D, CH), jnp.float32),
            pltpu.VMEM((D, 64), jnp.float32),
            pltpu.VMEM((SR, 128), jnp.float32),
            pltpu.VMEM((SR,), _I32),
            pltpu.VMEM((1024,), _I32),
            pltpu.SemaphoreType.DMA,
            pltpu.SemaphoreType.DMA,
            pltpu.SemaphoreType.DMA,
            pltpu.SemaphoreType.DMA,
        ],
        compiler_params=pltpu.CompilerParams(
            use_tc_tiling_on_sc=True, needs_layout_passes=False
        ),
    )
    def scan_kernel(
        idx_hbm, tab_hbm, out_hbm,
        ibuf0, ibuf1, hits_u, hits_pos, buf0, buf1, tailbuf,
        stage, stage_pos, loc, sem_i, sem0, sem1, sem_out,
    ):
        wid = lax.axis_index("s") * NC + lax.axis_index("c")
        iota = lax.broadcasted_iota(_I32, (L,), 0)
        neg1 = jnp.full((L,), -1, _I32)

        for r in range(SR // L):
            stage_pos[pl.ds(r * L, L)] = neg1

        lo = wid * MAIN
        far = jnp.asarray(2**30, _I32)
        EW = CH // 4
        elo = jnp.where(wid >= NW - 4, extra_lo + (wid - (NW - 4)) * EW, far)
        tlo = jnp.where(wid == NW - 5, tail_lo, far)

        pltpu.async_copy(tab_hbm.at[:, pl.ds(lo, CH)], buf0, sem0)
        pltpu.async_copy(tab_hbm.at[:, pl.ds(lo + CH, CH)], buf1, sem1)

        nh = _I32(0)
        descs = [None, None]
        descs[0] = pltpu.async_copy(idx_hbm.at[pl.ds(0, 1024)], ibuf0, sem_i)
        for ib in range(NIB):
            cur = ibuf0 if ib % 2 == 0 else ibuf1
            nxt = ibuf1 if ib % 2 == 0 else ibuf0
            descs[ib % 2].wait()
            if ib + 1 < NIB:
                descs[(ib + 1) % 2] = pltpu.async_copy(
                    idx_hbm.at[pl.ds((ib + 1) * 1024, 1024)], nxt, sem_i
                )

            def _scan(j, cursor, cur=cur, ib=ib):
                v = cur[pl.ds(j * L, L)]
                pos = iota + (ib * 1024) + j * L
                m = (
                    ((v >= lo) & (v < lo + MAIN))
                    | ((v >= elo) & (v < elo + EW))
                    | (v >= tlo)
                )
                offs = plsc.cumsum(jnp.where(m, 1, 0).astype(_I32))
                slots = cursor + offs - 1
                plsc.store_scatter(hits_u, [slots], v, mask=m)
                plsc.store_scatter(hits_pos, [slots], pos, mask=m)
                return cursor + offs[15]

            nh = lax.fori_loop(0, 1024 // L, _scan, nh)

        plsc.store_scatter(hits_u, [nh + iota], neg1, mask=(nh + iota) < B)

        nblocks = (nh + (L - 1)) // L

        def _flush():
            pltpu.async_copy(
                stage,
                out_hbm.at[plsc.Indices(stage_pos, ignored_value=-1)],
                sem_out,
            ).wait()
            for r in range(SR // L):
                stage_pos[pl.ds(r * L, L)] = neg1

        LOC = 1024

        def _extract(buf, c0u, width, cursor):
            def _passA(P):
                def _blkA(j, cnt):
                    vu = hits_u[pl.ds(j * L, L)]
                    m = (vu >= c0u) & (vu < c0u + width)
                    offs = plsc.cumsum(jnp.where(m, 1, 0).astype(_I32))
                    tot = offs[15]

                    @pl.when(tot > 0)
                    def _():
                        vp = hits_pos[pl.ds(j * L, L)]
                        packed = jnp.where(m, vu - c0u, 0) | (vp << 10)
                        gr = cnt + offs - 1
                        sel = m & (gr >= P) & (gr < P + LOC)
                        plsc.store_scatter(loc, [gr - P], packed, mask=sel)

                    return cnt + tot

                return lax.fori_loop(0, nblocks, _blkA, jnp.int32(0))

            def _process(n_this, cursor):
                def _blkB(j, cur):
                    packed = loc[pl.ds(j * L, L)]
                    m2 = (iota + j * L) < n_this
                    vul = jnp.where(m2, packed & (LOC - 1), 0)
                    vp = packed >> 10
                    slots = cur + iota
                    for f in range(D):
                        fv = jnp.full((L,), f, _I32)
                        val = plsc.load_gather(buf, [fv, vul])
                        plsc.store_scatter(stage, [slots, fv], val, mask=m2)
                    plsc.store_scatter(stage_pos, [slots], vp, mask=m2)
                    cur2 = cur + jnp.minimum(n_this - j * L, L)

                    @pl.when(cur2 >= FLUSH_AT)
                    def _():
                        _flush()

                    return jnp.where(cur2 >= FLUSH_AT, 0, cur2)

                return lax.fori_loop(0, (n_this + (L - 1)) // L, _blkB, cursor)

            n_chunk = _passA(jnp.int32(0))
            cursor = _process(jnp.minimum(n_chunk, LOC), cursor)

            def _more(st):
                P, cur = st
                _passA(P)
                cur = _process(jnp.minimum(n_chunk - P, LOC), cur)
                return (P + LOC, cur)

            _, cursor = lax.while_loop(
                lambda st: st[0] < n_chunk, _more, (jnp.int32(LOC), cursor)
            )
            return cursor

        def _pair(k, cursor):
            ca = lo + (2 * k) * CH
            pltpu.make_async_copy(tab_hbm.at[:, pl.ds(ca, CH)], buf0, sem0).wait()
            cursor = _extract(buf0, ca, CH, cursor)

            @pl.when(2 * k + 2 < CPW)
            def _():
                pltpu.async_copy(
                    tab_hbm.at[:, pl.ds(ca + 2 * CH, CH)], buf0, sem0
                )

            pltpu.make_async_copy(
                tab_hbm.at[:, pl.ds(ca + CH, CH)], buf1, sem1
            ).wait()
            cursor = _extract(buf1, ca + CH, CH, cursor)

            @pl.when(2 * k + 3 < CPW)
            def _():
                pltpu.async_copy(
                    tab_hbm.at[:, pl.ds(ca + 3 * CH, CH)], buf1, sem1
                )

            return cursor

        cursor = lax.fori_loop(0, CPW // 2, _pair, jnp.int32(0))
        if CPW % 2 == 1:
            clast = lo + (CPW - 1) * CH
            pltpu.make_async_copy(
                tab_hbm.at[:, pl.ds(clast, CH)], buf0, sem0
            ).wait()
            cursor = _extract(buf0, clast, CH, cursor)

        @pl.when(wid >= NW - 4)
        def _():
            pltpu.sync_copy(tab_hbm.at[:, pl.ds(elo, EW)], buf1.at[:, :EW])

        cursor = _extract(buf1, elo, EW, cursor)

        @pl.when(wid == NW - 5)
        def _():
            pltpu.sync_copy(
                tab_hbm.at[:, pl.ds(tail_lo, tail)], tailbuf.at[:, :tail]
            )

        cursor = _extract(tailbuf, tlo, tail, cursor)

        _flush()

    out128 = scan_kernel(user_indices.astype(_I32), tab_t)
    return out128[:, :D]
